# baseline (speedup 1.0000x reference)
import functools

import jax
import jax.numpy as jnp
from jax import lax
from jax.experimental import pallas as pl
from jax.experimental.pallas import tpu as pltpu
from jax.experimental.pallas import tpu_sc as plsc

TOPK = 8
POOL = 64
TM = 1024
SUB = 8


def _vw_body(value_pool_ref, w2_ref, vw_ref):
    vw_ref[...] = lax.dot_general(
        value_pool_ref[...], w2_ref[...],
        (((1,), (1,)), ((), ())),
        preferred_element_type=jnp.float32,
    )


def _build_vw(value_pool, fc_w, emb_dim):
    return pl.pallas_call(
        _vw_body,
        grid=(TOPK,),
        in_specs=[
            pl.BlockSpec((POOL, emb_dim), lambda j: (0, 0)),
            pl.BlockSpec((emb_dim, emb_dim), lambda j: (0, j + 1)),
        ],
        out_specs=pl.BlockSpec((POOL, emb_dim), lambda j: (j, 0)),
        out_shape=jax.ShapeDtypeStruct((TOPK * POOL, emb_dim), jnp.float32),
    )(value_pool, fc_w)


def _main_body(hn_ref, nrm_ref, kn_ref, w1_ref, b_ref,
               out_ref, idx_ref, sim_ref, sim_acc):
    i = pl.program_id(0)
    n_steps = pl.num_programs(0)

    hn = hn_ref[...]
    kn = kn_ref[...]

    w = lax.dot_general(hn, kn, (((1,), (1,)), ((), ())),
                        preferred_element_type=jnp.float32)

    dense = lax.dot_general(hn.astype(jnp.bfloat16), w1_ref[...],
                            (((1,), (1,)), ((), ())),
                            preferred_element_type=jnp.float32)

    iota = lax.broadcasted_iota(jnp.int32, (TM, POOL), 1).astype(jnp.float32)
    topsum = jnp.zeros((TM, 1), jnp.float32)
    for j in range(TOPK):
        m = jnp.max(w, axis=1, keepdims=True)
        sel = w == m
        idx_f = jnp.min(jnp.where(sel, iota, float(POOL)),
                        axis=1, keepdims=True)
        oh = (iota == idx_f)
        w = jnp.where(oh, -jnp.inf, w)
        topsum = topsum + m
        idx_ref[j, :] = (idx_f[:, 0] + float(POOL) * j).astype(jnp.int32)

    out_ref[...] = dense * nrm_ref[...] + b_ref[...]

    @pl.when(i == 0)
    def _():
        sim_acc[0] = 0.0
    sim_acc[0] += jnp.sum(topsum)

    @pl.when(i == n_steps - 1)
    def _():
        sim_ref[...] = jnp.reshape(sim_acc[0], (1, 1))


def _sc_pool_accumulate(vw, idx_flat, dense, n_tokens, emb_dim):
    mesh = plsc.VectorSubcoreMesh(core_axis_name="c", subcore_axis_name="s")
    info = plsc.get_sparse_core_info()
    nc, ns = info.num_cores, info.num_subcores
    nw = nc * ns
    per_w = n_tokens // nw
    n_chunks = per_w // SUB

    @functools.partial(
        pl.kernel, mesh=mesh,
        out_type=jax.ShapeDtypeStruct((n_tokens, emb_dim), jnp.float32),
        scratch_types=[
            pltpu.VMEM((TOPK * per_w,), jnp.int32),
            pltpu.VMEM((TOPK * SUB, emb_dim), jnp.float32),
            pltpu.VMEM((SUB, emb_dim), jnp.float32),
            pltpu.SemaphoreType.DMA,
        ],
    )
    def k(vw_hbm, idx_hbm, dense_hbm, out_hbm, idxv, rows, acc, sem):
        wid = lax.axis_index("s") * nc + lax.axis_index("c")
        base = wid * per_w

        for j in range(TOPK):
            pltpu.sync_copy(idx_hbm.at[pl.ds(j * n_tokens + base, per_w)],
                            idxv.at[pl.ds(j * per_w, per_w)])

        def chunk_body(cstep, carry):
            tok = base + cstep * SUB
            copies = [
                pltpu.async_copy(dense_hbm.at[pl.ds(tok, SUB), :], acc, sem)
            ] + [
                pltpu.async_copy(
                    vw_hbm.at[idxv.at[pl.ds(j * per_w + cstep * SUB, SUB)]],
                    rows.at[pl.ds(j * SUB, SUB), :], sem)
                for j in range(TOPK)
            ]
            for c in copies:
                c.wait()
            for t in range(SUB):
                def add_body(kk, c2, _t=t):
                    s = kk * 64
                    for u in range(4):
                        sl = pl.ds(s + u * 16, 16)
                        x = rows[_t, sl]
                        for j in range(1, TOPK):
                            x = x + rows[j * SUB + _t, sl]
                        plsc.addupdate(acc.at[_t, sl], x)
                    return c2
                lax.fori_loop(0, emb_dim // 64, add_body, 0)
            pltpu.sync_copy(acc, out_hbm.at[pl.ds(tok, SUB), :])
            return carry

        lax.fori_loop(0, n_chunks, chunk_body, 0)

    return k(vw, idx_flat, dense)


def kernel(hidden_states, key_pool, value_pool, fc_w, fc_b):
    B, T, D = hidden_states.shape
    O = fc_w.shape[0]
    N = B * T

    def _l2n(x):
        n = jnp.maximum(jnp.sqrt(jnp.sum(x * x, axis=-1, keepdims=True)), 1e-12)
        return x / n, n

    hn, nrm = _l2n(hidden_states)
    hn2 = hn.reshape(N, D)
    nrm2 = nrm.reshape(N, 1)
    kn, _ = _l2n(key_pool)
    w1b = fc_w[:, :D].astype(jnp.bfloat16)
    vw = _build_vw(value_pool, fc_w, D)

    grid = (N // TM,)
    dense2, idx, sim = pl.pallas_call(
        _main_body,
        grid=grid,
        in_specs=[
            pl.BlockSpec((TM, D), lambda i: (i, 0)),
            pl.BlockSpec((TM, 1), lambda i: (i, 0)),
            pl.BlockSpec((POOL, D), lambda i: (0, 0)),
            pl.BlockSpec((O, D), lambda i: (0, 0)),
            pl.BlockSpec((1, O), lambda i: (0, 0)),
        ],
        out_specs=[
            pl.BlockSpec((TM, O), lambda i: (i, 0)),
            pl.BlockSpec((TOPK, TM), lambda i: (0, i)),
            pl.BlockSpec((1, 1), lambda i: (0, 0)),
        ],
        out_shape=[
            jax.ShapeDtypeStruct((N, O), jnp.float32),
            jax.ShapeDtypeStruct((TOPK, N), jnp.int32),
            jax.ShapeDtypeStruct((1, 1), jnp.float32),
        ],
        scratch_shapes=[pltpu.SMEM((1,), jnp.float32)],
    )(hn2, nrm2, kn, w1b, fc_b.reshape(1, O))

    idx_flat = idx.reshape(TOPK * N)
    out2 = _sc_pool_accumulate(vw, idx_flat, dense2, N, O)

    out = out2.reshape(B, T, O)
    sim_loss = sim[0, 0] / B
    return (out, sim_loss)
